# Initial kernel scaffold; baseline (speedup 1.0000x reference)
import jax
import jax.numpy as jnp
from jax.experimental import pallas as pl


def kernel(x, edge_index, edge_weight, emb, mlp_W1, mlp_b1, mlp_W2, mlp_b2, lin1_W, lin2_W, lin2_b, out1_W, out1_b, out2_W, out2_b):
    raise NotImplementedError("write your pallas kernel here")


# speedup vs baseline: 1.1623x; 1.1623x over previous
import functools

import numpy as np
import jax
import jax.numpy as jnp
from jax import lax
from jax.experimental import pallas as pl
from jax.experimental.pallas import tpu as pltpu
from jax.experimental.pallas import tpu_sc as plsc

_N = 10000
_E = 320000
_H = 128
_NG = 50
_NGP = 64
_CUT = 5.0
_LOG2 = float(np.log(2.0))
_COEFF = -0.5 / (_CUT / _NG) ** 2
_PI = float(np.pi)

def _offsets():
    col = lax.broadcasted_iota(jnp.int32, (1, _NGP), 1)
    colf = col.astype(_F32)
    return jnp.where(col < _NG, colf * (_CUT / (_NG - 1)), 0.0)

_NC = 2
_NS = 16
_NW = _NC * _NS
_CH = 80

_F32 = jnp.float32


def _ssp(v):
    return jax.nn.softplus(v) - _LOG2


def _edge_geom(ew):
    d = jnp.sqrt(jnp.sum(ew * ew, axis=1, keepdims=True))
    offs = _offsets()
    rbf = jnp.exp(_COEFF * (d - offs) ** 2)
    C = 0.5 * (jnp.cos(d * (_PI / _CUT)) + 1.0) * (d <= _CUT)
    return d, rbf, C


def _mm(a, b):
    return jnp.dot(a.astype(jnp.bfloat16), b.astype(jnp.bfloat16),
                   preferred_element_type=_F32)


def _mm_t(a, b):
    return lax.dot_general(a, b, (((1,), (1,)), ((), ())),
                           preferred_element_type=_F32,
                           precision=lax.Precision.HIGHEST)


def _q_body(h_ref, w_ref, o_ref):
    o_ref[...] = _mm(h_ref[...], w_ref[...])


def _tc_q(h, w):
    BN = 2000
    return pl.pallas_call(
        _q_body,
        grid=(_N // BN,),
        in_specs=[pl.BlockSpec((BN, _H), lambda b: (b, 0)),
                  pl.BlockSpec((_H, _H), lambda b: (0, 0))],
        out_specs=pl.BlockSpec((BN, _H), lambda b: (b, 0)),
        out_shape=jax.ShapeDtypeStruct((_N, _H), _F32),
    )(h, w)


def _edge_fwd_body(ew_ref, p_ref, w1_ref, b1_ref, w2_ref, b2_ref, msg_ref):
    d, rbf, C = _edge_geom(ew_ref[...])
    A = _mm(rbf, w1_ref[...]) + b1_ref[...]
    B = _mm(_ssp(A), w2_ref[...]) + b2_ref[...]
    msg_ref[...] = (B * C) * p_ref[...]


def _tc_edge_fwd(ew, p, w1, b1, w2, b2):
    BE = 1600
    g = _E // BE
    return pl.pallas_call(
        _edge_fwd_body,
        grid=(g,),
        in_specs=[pl.BlockSpec((BE, 3), lambda b: (b, 0)),
                  pl.BlockSpec((BE, _H), lambda b: (b, 0)),
                  pl.BlockSpec((_NGP, _H), lambda b: (0, 0)),
                  pl.BlockSpec((1, _H), lambda b: (0, 0)),
                  pl.BlockSpec((_H, _H), lambda b: (0, 0)),
                  pl.BlockSpec((1, _H), lambda b: (0, 0))],
        out_specs=pl.BlockSpec((BE, _H), lambda b: (b, 0)),
        out_shape=jax.ShapeDtypeStruct((_E, _H), _F32),
    )(ew, p, w1, b1, w2, b2)


def _node_fwd_body(h_ref, a0_ref, a1_ref, w_ref, b_ref, hn_ref, t_ref):
    t = _mm(a0_ref[...] + a1_ref[...], w_ref[...]) + b_ref[...]
    t_ref[...] = t
    hn_ref[...] = h_ref[...] + _ssp(t)


def _tc_node_fwd(h, a0, a1, w, b):
    BN = 2000
    blk = pl.BlockSpec((BN, _H), lambda i: (i, 0))
    wblk = pl.BlockSpec((_H, _H), lambda i: (0, 0))
    bblk = pl.BlockSpec((1, _H), lambda i: (0, 0))
    return pl.pallas_call(
        _node_fwd_body,
        grid=(_N // BN,),
        in_specs=[blk, blk, blk, wblk, bblk],
        out_specs=[blk, blk],
        out_shape=[jax.ShapeDtypeStruct((_N, _H), _F32),
                   jax.ShapeDtypeStruct((_N, _H), _F32)],
    )(h, a0, a1, w, b)


def _readout_body(h_ref, w1_ref, b1_ref, w2r_ref, b2_ref, g_ref, e_ref):
    h = h_ref[...]
    y1 = _mm(h, w1_ref[...]) + b1_ref[...]
    w2r = w2r_ref[...]
    zb = _ssp(y1).astype(jnp.bfloat16).astype(_F32)
    wb = w2r.astype(jnp.bfloat16).astype(_F32)
    s = jnp.sum(zb * wb) + h.shape[0] * b2_ref[0, 0]

    @pl.when(pl.program_id(0) == 0)
    def _():
        e_ref[...] = jnp.zeros_like(e_ref)

    e_ref[...] += jnp.full(e_ref.shape, s, _F32)
    g_ref[...] = _mm_t(jax.nn.sigmoid(y1) * w2r, w1_ref[...])


def _tc_readout(h, out1_W, out1_b, out2_W, out2_b):
    BN = 2000
    return pl.pallas_call(
        _readout_body,
        grid=(_N // BN,),
        in_specs=[pl.BlockSpec((BN, _H), lambda i: (i, 0)),
                  pl.BlockSpec((_H, 64), lambda i: (0, 0)),
                  pl.BlockSpec((1, 64), lambda i: (0, 0)),
                  pl.BlockSpec((1, 64), lambda i: (0, 0)),
                  pl.BlockSpec((1, 128), lambda i: (0, 0))],
        out_specs=[pl.BlockSpec((BN, _H), lambda i: (i, 0)),
                   pl.BlockSpec((8, 128), lambda i: (0, 0))],
        out_shape=[jax.ShapeDtypeStruct((_N, _H), _F32),
                   jax.ShapeDtypeStruct((8, 128), _F32)],
    )(h, out1_W, out1_b.reshape(1, 64), out2_W.reshape(1, 64),
      jnp.broadcast_to(out2_b.reshape(1, 1), (1, 128)))


def _bwd_pre_body(g_ref, t_ref, w_ref, o_ref):
    o_ref[...] = _mm_t(g_ref[...] * jax.nn.sigmoid(t_ref[...]), w_ref[...])


def _tc_node_bwd_pre(G, t, w):
    BN = 2000
    blk = pl.BlockSpec((BN, _H), lambda i: (i, 0))
    return pl.pallas_call(
        _bwd_pre_body,
        grid=(_N // BN,),
        in_specs=[blk, blk, pl.BlockSpec((_H, _H), lambda i: (0, 0))],
        out_specs=blk,
        out_shape=jax.ShapeDtypeStruct((_N, _H), _F32),
    )(G, t, w)


def _edge_bwd_body(ew_ref, p_ref, dm_ref, dprev_ref,
                   w1_ref, b1_ref, w2_ref, b2_ref, dp_ref, dout_ref):
    ew = ew_ref[...]
    d, rbf, C = _edge_geom(ew)
    w1 = w1_ref[...]
    w2 = w2_ref[...]
    A = _mm(rbf, w1) + b1_ref[...]
    sigA = jax.nn.sigmoid(A)
    B = _mm(_ssp(A), w2) + b2_ref[...]
    Wf = B * C
    dm = dm_ref[...]
    p = p_ref[...]
    dWf = dm * p
    dp_ref[...] = dm * Wf
    dC = jnp.sum(dWf * B, axis=1, keepdims=True)
    dS = _mm_t(dWf * C, w2)
    drbf = _mm_t(dS * sigA, w1)
    offs = _offsets()
    ddr = jnp.sum(drbf * rbf * (2.0 * _COEFF) * (d - offs),
                  axis=1, keepdims=True)
    dCd = (-0.5 * _PI / _CUT) * jnp.sin(d * (_PI / _CUT)) * (d <= _CUT)
    dd = ddr + dC * dCd
    dout_ref[...] = dprev_ref[...] + dd * ew / d


def _tc_edge_bwd(ew, p, dm, dprev, w1, b1, w2, b2):
    BE = 1600
    g = _E // BE
    eblk = pl.BlockSpec((BE, _H), lambda b: (b, 0))
    vblk = pl.BlockSpec((BE, 3), lambda b: (b, 0))
    return pl.pallas_call(
        _edge_bwd_body,
        grid=(g,),
        in_specs=[vblk, eblk, eblk, vblk,
                  pl.BlockSpec((_NGP, _H), lambda b: (0, 0)),
                  pl.BlockSpec((1, _H), lambda b: (0, 0)),
                  pl.BlockSpec((_H, _H), lambda b: (0, 0)),
                  pl.BlockSpec((1, _H), lambda b: (0, 0))],
        out_specs=[eblk, vblk],
        out_shape=[jax.ShapeDtypeStruct((_E, _H), _F32),
                   jax.ShapeDtypeStruct((_E, 3), _F32)],
    )(ew, p, dm, dprev, w1, b1, w2, b2)


def _bwd_post_body(g_ref, a0_ref, a1_ref, w_ref, o_ref):
    o_ref[...] = g_ref[...] + _mm_t(a0_ref[...] + a1_ref[...], w_ref[...])


def _tc_node_bwd_post(G, a0, a1, w):
    BN = 2000
    blk = pl.BlockSpec((BN, _H), lambda i: (i, 0))
    return pl.pallas_call(
        _bwd_post_body,
        grid=(_N // BN,),
        in_specs=[blk, blk, blk, pl.BlockSpec((_H, _H), lambda i: (0, 0))],
        out_specs=blk,
        out_shape=jax.ShapeDtypeStruct((_N, _H), _F32),
    )(G, a0, a1, w)


def _fpad_body(df_ref, pos_ref, neg_ref):
    df = df_ref[...]
    z = jnp.zeros((df.shape[0], 125), _F32)
    pos = jnp.concatenate([df, z], axis=1)
    pos_ref[...] = pos
    neg_ref[...] = -pos


def _tc_force_pad(diff):
    BE = 3200
    return pl.pallas_call(
        _fpad_body,
        grid=(_E // BE,),
        in_specs=[pl.BlockSpec((BE, 3), lambda b: (b, 0))],
        out_specs=[pl.BlockSpec((BE, _H), lambda b: (b, 0)),
                   pl.BlockSpec((BE, _H), lambda b: (b, 0))],
        out_shape=[jax.ShapeDtypeStruct((_E, _H), _F32),
                   jax.ShapeDtypeStruct((_E, _H), _F32)],
    )(diff)


def _combine_body(a_ref, b_ref, o_ref):
    o_ref[...] = a_ref[...] + b_ref[...]


def _tc_combine(a, b):
    BN = 2000
    D = a.shape[1]
    blk = pl.BlockSpec((BN, D), lambda i: (i, 0))
    return pl.pallas_call(
        _combine_body,
        grid=(_N // BN,),
        in_specs=[blk, blk],
        out_specs=blk,
        out_shape=jax.ShapeDtypeStruct((_N, D), _F32),
    )(a, b)


def _sc_gather(table, idx):
    B = idx.shape[0]
    D = table.shape[1]
    per_w = B // _NW
    n_chunks = per_w // _CH
    mesh = plsc.VectorSubcoreMesh(core_axis_name="c", subcore_axis_name="s")

    @functools.partial(
        pl.kernel,
        out_type=jax.ShapeDtypeStruct((B, D), _F32),
        mesh=mesh,
        scratch_types=[pltpu.VMEM((_CH,), jnp.int32),
                       pltpu.VMEM((_CH, D), _F32),
                       pltpu.SemaphoreType.DMA],
    )
    def k(table_hbm, idx_hbm, out_hbm, idx_v, rows_v, sem):
        wid = lax.axis_index("s") * _NC + lax.axis_index("c")
        base = wid * per_w

        def body(c, _):
            off = base + c * _CH
            pltpu.sync_copy(idx_hbm.at[pl.ds(off, _CH)], idx_v)
            pltpu.async_copy(table_hbm.at[idx_v], rows_v, sem).wait()
            pltpu.sync_copy(rows_v, out_hbm.at[pl.ds(off, _CH)])
            return 0

        lax.fori_loop(0, n_chunks, body, 0)

    return k(table, idx)


def _sc_scatter_add(idx, vals, zeros_tab):
    B, D = vals.shape
    n_rows = zeros_tab.shape[0]
    per_w = B // _NW
    n_chunks = per_w // _CH
    rows_per_s = n_rows // _NS
    assert rows_per_s % 8 == 0 and n_rows % _NS == 0
    mesh = plsc.VectorSubcoreMesh(core_axis_name="c", subcore_axis_name="s")

    @functools.partial(
        pl.kernel,
        out_type=jax.ShapeDtypeStruct((2, n_rows, D), _F32),
        mesh=mesh,
        scratch_types=[pltpu.VMEM((_CH,), jnp.int32),
                       pltpu.VMEM((_CH, D), _F32),
                       pltpu.VMEM_SHARED((n_rows, D), _F32),
                       pltpu.SemaphoreType.DMA],
    )
    def k(idx_hbm, vals_hbm, zeros_hbm, out_hbm, idx_v, val_v, acc_sh, sem):
        c = lax.axis_index("c")
        s = lax.axis_index("s")
        wid = s * _NC + c
        base = wid * per_w

        @pl.when(s == 0)
        def _():
            pltpu.sync_copy(zeros_hbm, acc_sh)

        plsc.subcore_barrier()

        def body(ch, _):
            off = base + ch * _CH
            pltpu.sync_copy(idx_hbm.at[pl.ds(off, _CH)], idx_v)
            pltpu.sync_copy(vals_hbm.at[pl.ds(off, _CH)], val_v)
            pltpu.sync_copy(val_v, acc_sh.at[idx_v], add=True)
            return 0

        lax.fori_loop(0, n_chunks, body, 0)
        plsc.subcore_barrier()
        r0 = s * rows_per_s
        pltpu.sync_copy(acc_sh.at[pl.ds(r0, rows_per_s)],
                        out_hbm.at[c, pl.ds(r0, rows_per_s)])

    return k(idx, vals, zeros_tab)


def kernel(x, edge_index, edge_weight, emb, mlp_W1, mlp_b1, mlp_W2, mlp_b2,
           lin1_W, lin2_W, lin2_b, out1_W, out1_b, out2_W, out2_b):
    i_idx = edge_index[0].astype(jnp.int32)
    j_idx = edge_index[1].astype(jnp.int32)
    ew = edge_weight.astype(_F32)

    npad = _NW * _CH * ((_N + _NW * _CH - 1) // (_NW * _CH))
    xp = jnp.concatenate(
        [x.astype(jnp.int32), jnp.zeros((npad - _N,), jnp.int32)])
    h = _sc_gather(emb.astype(_F32), xp)[:_N]

    W1p = jnp.pad(mlp_W1.astype(_F32), ((0, 0), (0, _NGP - _NG), (0, 0)))
    _NP = 10240
    zerosN = jnp.zeros((_NP, _H), _F32)

    qs = []
    ts = []
    for l in range(3):
        q = _tc_q(h, lin1_W[l])
        p = _sc_gather(q, j_idx)
        msg = _tc_edge_fwd(ew, p, W1p[l], mlp_b1[l].reshape(1, _H),
                           mlp_W2[l], mlp_b2[l].reshape(1, _H))
        agg2 = _sc_scatter_add(i_idx, msg, zerosN)
        h, t = _tc_node_fwd(h, agg2[0, :_N], agg2[1, :_N], lin2_W[l],
                            lin2_b[l].reshape(1, _H))
        qs.append(q)
        ts.append(t)

    G, e_acc = _tc_readout(h, out1_W, out1_b, out2_W, out2_b)
    total_energy = e_acc[0, 0]

    diff = jnp.zeros((_E, 3), _F32)
    for l in (2, 1, 0):
        dagg = _tc_node_bwd_pre(G, ts[l], lin2_W[l])
        dmsg = _sc_gather(dagg, i_idx)
        p = _sc_gather(qs[l], j_idx)
        dp, diff = _tc_edge_bwd(ew, p, dmsg, diff, W1p[l],
                                mlp_b1[l].reshape(1, _H), mlp_W2[l],
                                mlp_b2[l].reshape(1, _H))
        if l > 0:
            dh2 = _sc_scatter_add(j_idx, dp, zerosN)
            G = _tc_node_bwd_post(G, dh2[0, :_N], dh2[1, :_N], lin1_W[l])

    pos, neg = _tc_force_pad(diff)
    vals = jnp.concatenate([pos, neg], axis=0)
    sidx = jnp.concatenate([i_idx, j_idx], axis=0)
    f2 = _sc_scatter_add(sidx, vals, zerosN)
    forces = _tc_combine(f2[0, :_N], f2[1, :_N])[:, :3]
    return (total_energy, forces)


# speedup vs baseline: 1.3503x; 1.1618x over previous
import functools

import numpy as np
import jax
import jax.numpy as jnp
from jax import lax
from jax.experimental import pallas as pl
from jax.experimental.pallas import tpu as pltpu
from jax.experimental.pallas import tpu_sc as plsc

_N = 10000
_E = 320000
_H = 128
_NG = 50
_NGP = 64
_CUT = 5.0
_LOG2 = float(np.log(2.0))
_COEFF = -0.5 / (_CUT / _NG) ** 2
_PI = float(np.pi)

def _offsets():
    col = lax.broadcasted_iota(jnp.int32, (1, _NGP), 1)
    colf = col.astype(_F32)
    return jnp.where(col < _NG, colf * (_CUT / (_NG - 1)), 0.0)

_NC = 2
_NS = 16
_NW = _NC * _NS
_CH = 80

_F32 = jnp.float32


def _ssp(v):
    return jax.nn.softplus(v) - _LOG2


def _edge_geom(ew):
    d = jnp.sqrt(jnp.sum(ew * ew, axis=1, keepdims=True))
    offs = _offsets()
    rbf = jnp.exp(_COEFF * (d - offs) ** 2)
    C = 0.5 * (jnp.cos(d * (_PI / _CUT)) + 1.0) * (d <= _CUT)
    return d, rbf, C


def _mm(a, b):
    return jnp.dot(a.astype(jnp.bfloat16), b.astype(jnp.bfloat16),
                   preferred_element_type=_F32)


def _mm_t(a, b):
    return lax.dot_general(a, b, (((1,), (1,)), ((), ())),
                           preferred_element_type=_F32,
                           precision=lax.Precision.HIGHEST)


def _q_body(h_ref, w_ref, o_ref):
    o_ref[...] = _mm(h_ref[...], w_ref[...])


def _tc_q(h, w):
    BN = 2000
    return pl.pallas_call(
        _q_body,
        grid=(_N // BN,),
        in_specs=[pl.BlockSpec((BN, _H), lambda b: (b, 0)),
                  pl.BlockSpec((_H, _H), lambda b: (0, 0))],
        out_specs=pl.BlockSpec((BN, _H), lambda b: (b, 0)),
        out_shape=jax.ShapeDtypeStruct((_N, _H), _F32),
    )(h, w)


def _edge_fwd_body(ew_ref, p_ref, w1_ref, b1_ref, w2_ref, b2_ref, msg_ref):
    d, rbf, C = _edge_geom(ew_ref[...])
    A = _mm(rbf, w1_ref[...]) + b1_ref[...]
    B = _mm(_ssp(A), w2_ref[...]) + b2_ref[...]
    msg_ref[...] = (B * C) * p_ref[...]


def _tc_edge_fwd(ew, p, w1, b1, w2, b2):
    BE = 1600
    g = _E // BE
    return pl.pallas_call(
        _edge_fwd_body,
        grid=(g,),
        in_specs=[pl.BlockSpec((BE, 3), lambda b: (b, 0)),
                  pl.BlockSpec((BE, _H), lambda b: (b, 0)),
                  pl.BlockSpec((_NGP, _H), lambda b: (0, 0)),
                  pl.BlockSpec((1, _H), lambda b: (0, 0)),
                  pl.BlockSpec((_H, _H), lambda b: (0, 0)),
                  pl.BlockSpec((1, _H), lambda b: (0, 0))],
        out_specs=pl.BlockSpec((BE, _H), lambda b: (b, 0)),
        out_shape=jax.ShapeDtypeStruct((_E, _H), _F32),
    )(ew, p, w1, b1, w2, b2)


def _node_fwd_body(h_ref, a0_ref, a1_ref, w_ref, b_ref, hn_ref, t_ref):
    t = _mm(a0_ref[...] + a1_ref[...], w_ref[...]) + b_ref[...]
    t_ref[...] = t
    hn_ref[...] = h_ref[...] + _ssp(t)


def _tc_node_fwd(h, a0, a1, w, b):
    BN = 2000
    blk = pl.BlockSpec((BN, _H), lambda i: (i, 0))
    wblk = pl.BlockSpec((_H, _H), lambda i: (0, 0))
    bblk = pl.BlockSpec((1, _H), lambda i: (0, 0))
    return pl.pallas_call(
        _node_fwd_body,
        grid=(_N // BN,),
        in_specs=[blk, blk, blk, wblk, bblk],
        out_specs=[blk, blk],
        out_shape=[jax.ShapeDtypeStruct((_N, _H), _F32),
                   jax.ShapeDtypeStruct((_N, _H), _F32)],
    )(h, a0, a1, w, b)


def _readout_body(h_ref, w1_ref, b1_ref, w2r_ref, b2_ref, g_ref, e_ref):
    h = h_ref[...]
    y1 = _mm(h, w1_ref[...]) + b1_ref[...]
    w2r = w2r_ref[...]
    zb = _ssp(y1).astype(jnp.bfloat16).astype(_F32)
    wb = w2r.astype(jnp.bfloat16).astype(_F32)
    s = jnp.sum(zb * wb) + h.shape[0] * b2_ref[0, 0]

    @pl.when(pl.program_id(0) == 0)
    def _():
        e_ref[...] = jnp.zeros_like(e_ref)

    e_ref[...] += jnp.full(e_ref.shape, s, _F32)
    g_ref[...] = _mm_t(jax.nn.sigmoid(y1) * w2r, w1_ref[...])


def _tc_readout(h, out1_W, out1_b, out2_W, out2_b):
    BN = 2000
    return pl.pallas_call(
        _readout_body,
        grid=(_N // BN,),
        in_specs=[pl.BlockSpec((BN, _H), lambda i: (i, 0)),
                  pl.BlockSpec((_H, 64), lambda i: (0, 0)),
                  pl.BlockSpec((1, 64), lambda i: (0, 0)),
                  pl.BlockSpec((1, 64), lambda i: (0, 0)),
                  pl.BlockSpec((1, 128), lambda i: (0, 0))],
        out_specs=[pl.BlockSpec((BN, _H), lambda i: (i, 0)),
                   pl.BlockSpec((8, 128), lambda i: (0, 0))],
        out_shape=[jax.ShapeDtypeStruct((_N, _H), _F32),
                   jax.ShapeDtypeStruct((8, 128), _F32)],
    )(h, out1_W, out1_b.reshape(1, 64), out2_W.reshape(1, 64),
      jnp.broadcast_to(out2_b.reshape(1, 1), (1, 128)))


def _bwd_pre_body(g_ref, t_ref, w_ref, o_ref):
    o_ref[...] = _mm_t(g_ref[...] * jax.nn.sigmoid(t_ref[...]), w_ref[...])


def _tc_node_bwd_pre(G, t, w):
    BN = 2000
    blk = pl.BlockSpec((BN, _H), lambda i: (i, 0))
    return pl.pallas_call(
        _bwd_pre_body,
        grid=(_N // BN,),
        in_specs=[blk, blk, pl.BlockSpec((_H, _H), lambda i: (0, 0))],
        out_specs=blk,
        out_shape=jax.ShapeDtypeStruct((_N, _H), _F32),
    )(G, t, w)


def _edge_bwd_body(ew_ref, p_ref, dm_ref, dprev_ref,
                   w1_ref, b1_ref, w2_ref, b2_ref, dp_ref, dout_ref):
    ew = ew_ref[...]
    d, rbf, C = _edge_geom(ew)
    w1 = w1_ref[...]
    w2 = w2_ref[...]
    A = _mm(rbf, w1) + b1_ref[...]
    sigA = jax.nn.sigmoid(A)
    B = _mm(_ssp(A), w2) + b2_ref[...]
    Wf = B * C
    dm = dm_ref[...]
    p = p_ref[...]
    dWf = dm * p
    dp_ref[...] = dm * Wf
    dC = jnp.sum(dWf * B, axis=1, keepdims=True)
    dS = _mm_t(dWf * C, w2)
    drbf = _mm_t(dS * sigA, w1)
    offs = _offsets()
    ddr = jnp.sum(drbf * rbf * (2.0 * _COEFF) * (d - offs),
                  axis=1, keepdims=True)
    dCd = (-0.5 * _PI / _CUT) * jnp.sin(d * (_PI / _CUT)) * (d <= _CUT)
    dd = ddr + dC * dCd
    dout_ref[...] = dprev_ref[...] + dd * ew / d


def _tc_edge_bwd(ew, p, dm, dprev, w1, b1, w2, b2):
    BE = 1600
    g = _E // BE
    eblk = pl.BlockSpec((BE, _H), lambda b: (b, 0))
    vblk = pl.BlockSpec((BE, 3), lambda b: (b, 0))
    return pl.pallas_call(
        _edge_bwd_body,
        grid=(g,),
        in_specs=[vblk, eblk, eblk, vblk,
                  pl.BlockSpec((_NGP, _H), lambda b: (0, 0)),
                  pl.BlockSpec((1, _H), lambda b: (0, 0)),
                  pl.BlockSpec((_H, _H), lambda b: (0, 0)),
                  pl.BlockSpec((1, _H), lambda b: (0, 0))],
        out_specs=[eblk, vblk],
        out_shape=[jax.ShapeDtypeStruct((_E, _H), _F32),
                   jax.ShapeDtypeStruct((_E, 3), _F32)],
    )(ew, p, dm, dprev, w1, b1, w2, b2)


def _bwd_post_body(g_ref, a0_ref, a1_ref, w_ref, o_ref):
    o_ref[...] = g_ref[...] + _mm_t(a0_ref[...] + a1_ref[...], w_ref[...])


def _tc_node_bwd_post(G, a0, a1, w):
    BN = 2000
    blk = pl.BlockSpec((BN, _H), lambda i: (i, 0))
    return pl.pallas_call(
        _bwd_post_body,
        grid=(_N // BN,),
        in_specs=[blk, blk, blk, pl.BlockSpec((_H, _H), lambda i: (0, 0))],
        out_specs=blk,
        out_shape=jax.ShapeDtypeStruct((_N, _H), _F32),
    )(G, a0, a1, w)


def _fpad_body(df_ref, pos_ref, neg_ref):
    df = df_ref[...]
    z = jnp.zeros((df.shape[0], 125), _F32)
    pos = jnp.concatenate([df, z], axis=1)
    pos_ref[...] = pos
    neg_ref[...] = -pos


def _tc_force_pad(diff):
    BE = 3200
    return pl.pallas_call(
        _fpad_body,
        grid=(_E // BE,),
        in_specs=[pl.BlockSpec((BE, 3), lambda b: (b, 0))],
        out_specs=[pl.BlockSpec((BE, _H), lambda b: (b, 0)),
                   pl.BlockSpec((BE, _H), lambda b: (b, 0))],
        out_shape=[jax.ShapeDtypeStruct((_E, _H), _F32),
                   jax.ShapeDtypeStruct((_E, _H), _F32)],
    )(diff)


def _combine_body(a_ref, b_ref, o_ref):
    o_ref[...] = a_ref[...] + b_ref[...]


def _tc_combine(a, b):
    BN = 2000
    D = a.shape[1]
    blk = pl.BlockSpec((BN, D), lambda i: (i, 0))
    return pl.pallas_call(
        _combine_body,
        grid=(_N // BN,),
        in_specs=[blk, blk],
        out_specs=blk,
        out_shape=jax.ShapeDtypeStruct((_N, D), _F32),
    )(a, b)


def _sc_gather_multi(pairs):
    kp = len(pairs)
    B = pairs[0][1].shape[0]
    Ds = [t.shape[1] for t, _ in pairs]
    per_w = B // _NW
    n = per_w // _CH
    n_main = (n // 2) * 2
    mesh = plsc.VectorSubcoreMesh(core_axis_name="c", subcore_axis_name="s")

    scratch = []
    for D in Ds:
        scratch += [pltpu.VMEM((_CH,), jnp.int32),
                    pltpu.VMEM((_CH,), jnp.int32),
                    pltpu.VMEM((_CH, D), _F32),
                    pltpu.VMEM((_CH, D), _F32)]
        scratch += [pltpu.SemaphoreType.DMA] * 6

    @functools.partial(
        pl.kernel,
        out_type=[jax.ShapeDtypeStruct((B, D), _F32) for D in Ds],
        mesh=mesh,
        scratch_types=scratch,
    )
    def k(*refs):
        tabs = [refs[2 * p] for p in range(kp)]
        idxs = [refs[2 * p + 1] for p in range(kp)]
        outs = [refs[2 * kp + p] for p in range(kp)]
        sc0 = 2 * kp + kp
        ib = [(refs[sc0 + 10 * p], refs[sc0 + 10 * p + 1]) for p in range(kp)]
        rb = [(refs[sc0 + 10 * p + 2], refs[sc0 + 10 * p + 3]) for p in range(kp)]
        sa = [(refs[sc0 + 10 * p + 4], refs[sc0 + 10 * p + 5]) for p in range(kp)]
        sb = [(refs[sc0 + 10 * p + 6], refs[sc0 + 10 * p + 7]) for p in range(kp)]
        so = [(refs[sc0 + 10 * p + 8], refs[sc0 + 10 * p + 9]) for p in range(kp)]

        wid = lax.axis_index("s") * _NC + lax.axis_index("c")
        base = wid * per_w

        def drain_idx(p, b):
            pltpu.make_async_copy(idxs[p].at[pl.ds(base, _CH)],
                                  ib[p][b], sa[p][b]).wait()

        def drain_rows(p, b):
            pltpu.make_async_copy(tabs[p].at[pl.ds(0, _CH)],
                                  rb[p][b], sb[p][b]).wait()

        def drain_out(p, b):
            pltpu.make_async_copy(tabs[p].at[pl.ds(0, _CH)],
                                  rb[p][b], so[p][b]).wait()

        if n_main > 0:
            for p in range(kp):
                for b in range(2):
                    pltpu.async_copy(idxs[p].at[pl.ds(base + b * _CH, _CH)],
                                     ib[p][b], sa[p][b])

            def body(g, _):
                for b in range(2):
                    c = 2 * g + b
                    for p in range(kp):
                        drain_idx(p, b)

                        @pl.when(c >= 2)
                        def _():
                            drain_out(p, b)

                        pltpu.async_copy(tabs[p].at[ib[p][b]], rb[p][b],
                                         sb[p][b])
                for b in range(2):
                    c = 2 * g + b
                    off = base + c * _CH
                    for p in range(kp):
                        drain_rows(p, b)
                        pltpu.async_copy(rb[p][b],
                                         outs[p].at[pl.ds(off, _CH)],
                                         so[p][b])

                        @pl.when(c + 2 < n_main)
                        def _():
                            pltpu.async_copy(
                                idxs[p].at[pl.ds(off + 2 * _CH, _CH)],
                                ib[p][b], sa[p][b])
                return 0

            lax.fori_loop(0, n_main // 2, body, 0)
            for p in range(kp):
                for b in range(2):
                    drain_out(p, b)

        for c in range(n_main, n):
            off = base + c * _CH
            for p in range(kp):
                pltpu.sync_copy(idxs[p].at[pl.ds(off, _CH)], ib[p][0])
                pltpu.async_copy(tabs[p].at[ib[p][0]], rb[p][0],
                                 sb[p][0]).wait()
                pltpu.sync_copy(rb[p][0], outs[p].at[pl.ds(off, _CH)])

    flat = []
    for t, i in pairs:
        flat += [t, i]
    out = k(*flat)
    if not isinstance(out, (list, tuple)):
        out = [out]
    return list(out)


def _sc_gather(table, idx):
    return _sc_gather_multi([(table, idx)])[0]


def _sc_scatter_add(idx, vals, zeros_tab):
    B, D = vals.shape
    n_rows = zeros_tab.shape[0]
    per_w = B // _NW
    n_chunks = per_w // _CH
    rows_per_s = n_rows // _NS
    assert rows_per_s % 8 == 0 and n_rows % _NS == 0
    mesh = plsc.VectorSubcoreMesh(core_axis_name="c", subcore_axis_name="s")

    @functools.partial(
        pl.kernel,
        out_type=jax.ShapeDtypeStruct((2, n_rows, D), _F32),
        mesh=mesh,
        scratch_types=[pltpu.VMEM((_CH,), jnp.int32),
                       pltpu.VMEM((_CH,), jnp.int32),
                       pltpu.VMEM((_CH, D), _F32),
                       pltpu.VMEM((_CH, D), _F32),
                       pltpu.VMEM_SHARED((n_rows, D), _F32),
                       pltpu.SemaphoreType.DMA,
                       pltpu.SemaphoreType.DMA,
                       pltpu.SemaphoreType.DMA,
                       pltpu.SemaphoreType.DMA],
    )
    def k(idx_hbm, vals_hbm, zeros_hbm, out_hbm, i0, i1, v0, v1,
          acc_sh, sa0, sa1, sv0, sv1):
        c = lax.axis_index("c")
        s = lax.axis_index("s")
        wid = s * _NC + c
        base = wid * per_w
        ib = (i0, i1)
        vb = (v0, v1)
        sa = (sa0, sa1)
        sv = (sv0, sv1)

        @pl.when(s == 0)
        def _():
            pltpu.sync_copy(zeros_hbm, acc_sh)

        plsc.subcore_barrier()

        n_main = (n_chunks // 2) * 2
        if n_main > 0:
            for b in range(2):
                off = base + b * _CH
                pltpu.async_copy(idx_hbm.at[pl.ds(off, _CH)], ib[b], sa[b])
                pltpu.async_copy(vals_hbm.at[pl.ds(off, _CH)], vb[b], sv[b])

            def body(g, _):
                for b in range(2):
                    ch = 2 * g + b
                    off = base + ch * _CH
                    pltpu.make_async_copy(idx_hbm.at[pl.ds(base, _CH)],
                                          ib[b], sa[b]).wait()
                    pltpu.make_async_copy(vals_hbm.at[pl.ds(base, _CH)],
                                          vb[b], sv[b]).wait()
                    pltpu.sync_copy(vb[b], acc_sh.at[ib[b]], add=True)

                    @pl.when(ch + 2 < n_main)
                    def _():
                        pltpu.async_copy(
                            idx_hbm.at[pl.ds(off + 2 * _CH, _CH)],
                            ib[b], sa[b])
                        pltpu.async_copy(
                            vals_hbm.at[pl.ds(off + 2 * _CH, _CH)],
                            vb[b], sv[b])
                return 0

            lax.fori_loop(0, n_main // 2, body, 0)

        for ch in range(n_main, n_chunks):
            off = base + ch * _CH
            pltpu.sync_copy(idx_hbm.at[pl.ds(off, _CH)], ib[0])
            pltpu.sync_copy(vals_hbm.at[pl.ds(off, _CH)], vb[0])
            pltpu.sync_copy(vb[0], acc_sh.at[ib[0]], add=True)

        plsc.subcore_barrier()
        r0 = s * rows_per_s
        pltpu.sync_copy(acc_sh.at[pl.ds(r0, rows_per_s)],
                        out_hbm.at[c, pl.ds(r0, rows_per_s)])

    return k(idx, vals, zeros_tab)


def kernel(x, edge_index, edge_weight, emb, mlp_W1, mlp_b1, mlp_W2, mlp_b2,
           lin1_W, lin2_W, lin2_b, out1_W, out1_b, out2_W, out2_b):
    i_idx = edge_index[0].astype(jnp.int32)
    j_idx = edge_index[1].astype(jnp.int32)
    ew = edge_weight.astype(_F32)

    npad = _NW * _CH * ((_N + _NW * _CH - 1) // (_NW * _CH))
    xp = jnp.concatenate(
        [x.astype(jnp.int32), jnp.zeros((npad - _N,), jnp.int32)])
    h = _sc_gather(emb.astype(_F32), xp)[:_N]

    W1p = jnp.pad(mlp_W1.astype(_F32), ((0, 0), (0, _NGP - _NG), (0, 0)))
    _NP = 10240
    zerosN = jnp.zeros((_NP, _H), _F32)

    qs = []
    ts = []
    for l in range(3):
        q = _tc_q(h, lin1_W[l])
        p = _sc_gather(q, j_idx)
        msg = _tc_edge_fwd(ew, p, W1p[l], mlp_b1[l].reshape(1, _H),
                           mlp_W2[l], mlp_b2[l].reshape(1, _H))
        agg2 = _sc_scatter_add(i_idx, msg, zerosN)
        h, t = _tc_node_fwd(h, agg2[0, :_N], agg2[1, :_N], lin2_W[l],
                            lin2_b[l].reshape(1, _H))
        qs.append(q)
        ts.append(t)

    G, e_acc = _tc_readout(h, out1_W, out1_b, out2_W, out2_b)
    total_energy = e_acc[0, 0]

    diff = jnp.zeros((_E, 3), _F32)
    for l in (2, 1, 0):
        dagg = _tc_node_bwd_pre(G, ts[l], lin2_W[l])
        dmsg, p = _sc_gather_multi([(dagg, i_idx), (qs[l], j_idx)])
        dp, diff = _tc_edge_bwd(ew, p, dmsg, diff, W1p[l],
                                mlp_b1[l].reshape(1, _H), mlp_W2[l],
                                mlp_b2[l].reshape(1, _H))
        if l > 0:
            dh2 = _sc_scatter_add(j_idx, dp, zerosN)
            G = _tc_node_bwd_post(G, dh2[0, :_N], dh2[1, :_N], lin1_W[l])

    pos, neg = _tc_force_pad(diff)
    vals = jnp.concatenate([pos, neg], axis=0)
    sidx = jnp.concatenate([i_idx, j_idx], axis=0)
    f2 = _sc_scatter_add(sidx, vals, zerosN)
    forces = _tc_combine(f2[0, :_N], f2[1, :_N])[:, :3]
    return (total_energy, forces)


# speedup vs baseline: 1.4073x; 1.0422x over previous
import functools

import numpy as np
import jax
import jax.numpy as jnp
from jax import lax
from jax.experimental import pallas as pl
from jax.experimental.pallas import tpu as pltpu
from jax.experimental.pallas import tpu_sc as plsc

_N = 10000
_E = 320000
_H = 128
_NG = 50
_NGP = 64
_CUT = 5.0
_LOG2 = float(np.log(2.0))
_COEFF = -0.5 / (_CUT / _NG) ** 2
_PI = float(np.pi)

def _offsets():
    col = lax.broadcasted_iota(jnp.int32, (1, _NGP), 1)
    colf = col.astype(_F32)
    return jnp.where(col < _NG, colf * (_CUT / (_NG - 1)), 0.0)

_NC = 2
_NS = 16
_NW = _NC * _NS
_CH = 80

_F32 = jnp.float32


def _ssp(v):
    return jax.nn.softplus(v) - _LOG2


def _edge_geom(ew):
    d = jnp.sqrt(jnp.sum(ew * ew, axis=1, keepdims=True))
    offs = _offsets()
    rbf = jnp.exp(_COEFF * (d - offs) ** 2)
    C = 0.5 * (jnp.cos(d * (_PI / _CUT)) + 1.0) * (d <= _CUT)
    return d, rbf, C


def _mm(a, b):
    return jnp.dot(a.astype(jnp.bfloat16), b.astype(jnp.bfloat16),
                   preferred_element_type=_F32)


def _mm_t(a, b):
    return lax.dot_general(a, b, (((1,), (1,)), ((), ())),
                           preferred_element_type=_F32,
                           precision=lax.Precision.HIGHEST)


def _q_body(h_ref, w_ref, o_ref):
    o_ref[...] = _mm(h_ref[...], w_ref[...])


def _tc_q(h, w):
    BN = 2000
    return pl.pallas_call(
        _q_body,
        grid=(_N // BN,),
        in_specs=[pl.BlockSpec((BN, _H), lambda b: (b, 0)),
                  pl.BlockSpec((_H, _H), lambda b: (0, 0))],
        out_specs=pl.BlockSpec((BN, _H), lambda b: (b, 0)),
        out_shape=jax.ShapeDtypeStruct((_N, _H), _F32),
    )(h, w)


def _edge_fwd_body(ew_ref, p_ref, w1_ref, b1_ref, w2_ref, b2_ref, msg_ref):
    d, rbf, C = _edge_geom(ew_ref[...])
    A = _mm(rbf, w1_ref[...]) + b1_ref[...]
    B = _mm(_ssp(A), w2_ref[...]) + b2_ref[...]
    msg_ref[...] = (B * C) * p_ref[...]


def _tc_edge_fwd(ew, p, w1, b1, w2, b2):
    BE = 1600
    g = _E // BE
    return pl.pallas_call(
        _edge_fwd_body,
        grid=(g,),
        in_specs=[pl.BlockSpec((BE, 3), lambda b: (b, 0)),
                  pl.BlockSpec((BE, _H), lambda b: (b, 0)),
                  pl.BlockSpec((_NGP, _H), lambda b: (0, 0)),
                  pl.BlockSpec((1, _H), lambda b: (0, 0)),
                  pl.BlockSpec((_H, _H), lambda b: (0, 0)),
                  pl.BlockSpec((1, _H), lambda b: (0, 0))],
        out_specs=pl.BlockSpec((BE, _H), lambda b: (b, 0)),
        out_shape=jax.ShapeDtypeStruct((_E, _H), _F32),
    )(ew, p, w1, b1, w2, b2)


def _node_fwd_body(h_ref, a0_ref, a1_ref, w_ref, b_ref, hn_ref, t_ref):
    t = _mm(a0_ref[...] + a1_ref[...], w_ref[...]) + b_ref[...]
    t_ref[...] = t
    hn_ref[...] = h_ref[...] + _ssp(t)


def _tc_node_fwd(h, a0, a1, w, b):
    BN = 2000
    blk = pl.BlockSpec((BN, _H), lambda i: (i, 0))
    wblk = pl.BlockSpec((_H, _H), lambda i: (0, 0))
    bblk = pl.BlockSpec((1, _H), lambda i: (0, 0))
    return pl.pallas_call(
        _node_fwd_body,
        grid=(_N // BN,),
        in_specs=[blk, blk, blk, wblk, bblk],
        out_specs=[blk, blk],
        out_shape=[jax.ShapeDtypeStruct((_N, _H), _F32),
                   jax.ShapeDtypeStruct((_N, _H), _F32)],
    )(h, a0, a1, w, b)


def _readout_body(h_ref, w1_ref, b1_ref, w2r_ref, b2_ref, g_ref, e_ref):
    h = h_ref[...]
    y1 = _mm(h, w1_ref[...]) + b1_ref[...]
    w2r = w2r_ref[...]
    zb = _ssp(y1).astype(jnp.bfloat16).astype(_F32)
    wb = w2r.astype(jnp.bfloat16).astype(_F32)
    s = jnp.sum(zb * wb) + h.shape[0] * b2_ref[0, 0]

    @pl.when(pl.program_id(0) == 0)
    def _():
        e_ref[...] = jnp.zeros_like(e_ref)

    e_ref[...] += jnp.full(e_ref.shape, s, _F32)
    g_ref[...] = _mm_t(jax.nn.sigmoid(y1) * w2r, w1_ref[...])


def _tc_readout(h, out1_W, out1_b, out2_W, out2_b):
    BN = 2000
    return pl.pallas_call(
        _readout_body,
        grid=(_N // BN,),
        in_specs=[pl.BlockSpec((BN, _H), lambda i: (i, 0)),
                  pl.BlockSpec((_H, 64), lambda i: (0, 0)),
                  pl.BlockSpec((1, 64), lambda i: (0, 0)),
                  pl.BlockSpec((1, 64), lambda i: (0, 0)),
                  pl.BlockSpec((1, 128), lambda i: (0, 0))],
        out_specs=[pl.BlockSpec((BN, _H), lambda i: (i, 0)),
                   pl.BlockSpec((8, 128), lambda i: (0, 0))],
        out_shape=[jax.ShapeDtypeStruct((_N, _H), _F32),
                   jax.ShapeDtypeStruct((8, 128), _F32)],
    )(h, out1_W, out1_b.reshape(1, 64), out2_W.reshape(1, 64),
      jnp.broadcast_to(out2_b.reshape(1, 1), (1, 128)))


def _bwd_pre_body(g_ref, t_ref, w_ref, o_ref):
    o_ref[...] = _mm_t(g_ref[...] * jax.nn.sigmoid(t_ref[...]), w_ref[...])


def _tc_node_bwd_pre(G, t, w):
    BN = 2000
    blk = pl.BlockSpec((BN, _H), lambda i: (i, 0))
    return pl.pallas_call(
        _bwd_pre_body,
        grid=(_N // BN,),
        in_specs=[blk, blk, pl.BlockSpec((_H, _H), lambda i: (0, 0))],
        out_specs=blk,
        out_shape=jax.ShapeDtypeStruct((_N, _H), _F32),
    )(G, t, w)


def _edge_bwd_common(ew_ref, p_ref, dm_ref, dprev_ref,
                     w1_ref, b1_ref, w2_ref, b2_ref, dp_ref):
    ew = ew_ref[...]
    d, rbf, C = _edge_geom(ew)
    w1 = w1_ref[...]
    w2 = w2_ref[...]
    A = _mm(rbf, w1) + b1_ref[...]
    sigA = jax.nn.sigmoid(A)
    B = _mm(_ssp(A), w2) + b2_ref[...]
    Wf = B * C
    dm = dm_ref[...]
    p = p_ref[...]
    dWf = dm * p
    dp_ref[...] = dm * Wf
    dC = jnp.sum(dWf * B, axis=1, keepdims=True)
    dS = _mm_t(dWf * C, w2)
    drbf = _mm_t(dS * sigA, w1)
    offs = _offsets()
    ddr = jnp.sum(drbf * rbf * (2.0 * _COEFF) * (d - offs),
                  axis=1, keepdims=True)
    dCd = (-0.5 * _PI / _CUT) * jnp.sin(d * (_PI / _CUT)) * (d <= _CUT)
    dd = ddr + dC * dCd
    return dprev_ref[...] + dd * ew / d


def _edge_bwd_body(ew_ref, p_ref, dm_ref, dprev_ref,
                   w1_ref, b1_ref, w2_ref, b2_ref, dp_ref, dout_ref):
    dout_ref[...] = _edge_bwd_common(ew_ref, p_ref, dm_ref, dprev_ref,
                                     w1_ref, b1_ref, w2_ref, b2_ref, dp_ref)


def _edge_bwd_final_body(ew_ref, p_ref, dm_ref, dprev_ref,
                         w1_ref, b1_ref, w2_ref, b2_ref,
                         dp_ref, pos_ref, neg_ref):
    dnew = _edge_bwd_common(ew_ref, p_ref, dm_ref, dprev_ref,
                            w1_ref, b1_ref, w2_ref, b2_ref, dp_ref)
    z = jnp.zeros((dnew.shape[0], _H - 3), _F32)
    pos = jnp.concatenate([dnew, z], axis=1)
    pos_ref[...] = pos
    neg_ref[...] = -pos


def _tc_edge_bwd(ew, p, dm, dprev, w1, b1, w2, b2, final):
    BE = 1600
    g = _E // BE
    eblk = pl.BlockSpec((BE, _H), lambda b: (b, 0))
    vblk = pl.BlockSpec((BE, 3), lambda b: (b, 0))
    if final:
        body = _edge_bwd_final_body
        out_specs = [eblk, eblk, eblk]
        out_shape = [jax.ShapeDtypeStruct((_E, _H), _F32),
                     jax.ShapeDtypeStruct((_E, _H), _F32),
                     jax.ShapeDtypeStruct((_E, _H), _F32)]
    else:
        body = _edge_bwd_body
        out_specs = [eblk, vblk]
        out_shape = [jax.ShapeDtypeStruct((_E, _H), _F32),
                     jax.ShapeDtypeStruct((_E, 3), _F32)]
    return pl.pallas_call(
        body,
        grid=(g,),
        in_specs=[vblk, eblk, eblk, vblk,
                  pl.BlockSpec((_NGP, _H), lambda b: (0, 0)),
                  pl.BlockSpec((1, _H), lambda b: (0, 0)),
                  pl.BlockSpec((_H, _H), lambda b: (0, 0)),
                  pl.BlockSpec((1, _H), lambda b: (0, 0))],
        out_specs=out_specs,
        out_shape=out_shape,
    )(ew, p, dm, dprev, w1, b1, w2, b2)


def _h0q0_body(xb_ref, emb_ref, w_ref, h_ref, q_ref):
    lane = lax.broadcasted_iota(jnp.int32, xb_ref.shape, 1).astype(_F32)
    onehot = (xb_ref[...] == lane).astype(_F32)
    h = jnp.dot(onehot, emb_ref[...], preferred_element_type=_F32,
                precision=lax.Precision.HIGHEST)
    h_ref[...] = h
    q_ref[...] = _mm(h, w_ref[...])


def _tc_h0q0(xb, embp, w):
    BN = 2000
    blk = pl.BlockSpec((BN, _H), lambda i: (i, 0))
    wblk = pl.BlockSpec((_H, _H), lambda i: (0, 0))
    return pl.pallas_call(
        _h0q0_body,
        grid=(_N // BN,),
        in_specs=[blk, wblk, wblk],
        out_specs=[blk, blk],
        out_shape=[jax.ShapeDtypeStruct((_N, _H), _F32),
                   jax.ShapeDtypeStruct((_N, _H), _F32)],
    )(xb, embp, w)


def _bwd_post_body(g_ref, a0_ref, a1_ref, w_ref, o_ref):
    o_ref[...] = g_ref[...] + _mm_t(a0_ref[...] + a1_ref[...], w_ref[...])


def _tc_node_bwd_post(G, a0, a1, w):
    BN = 2000
    blk = pl.BlockSpec((BN, _H), lambda i: (i, 0))
    return pl.pallas_call(
        _bwd_post_body,
        grid=(_N // BN,),
        in_specs=[blk, blk, blk, pl.BlockSpec((_H, _H), lambda i: (0, 0))],
        out_specs=blk,
        out_shape=jax.ShapeDtypeStruct((_N, _H), _F32),
    )(G, a0, a1, w)


def _combine_body(a_ref, b_ref, o_ref):
    o_ref[...] = a_ref[...] + b_ref[...]


def _tc_combine(a, b):
    BN = 2000
    D = a.shape[1]
    blk = pl.BlockSpec((BN, D), lambda i: (i, 0))
    return pl.pallas_call(
        _combine_body,
        grid=(_N // BN,),
        in_specs=[blk, blk],
        out_specs=blk,
        out_shape=jax.ShapeDtypeStruct((_N, D), _F32),
    )(a, b)


def _sc_gather_multi(pairs):
    kp = len(pairs)
    B = pairs[0][1].shape[0]
    Ds = [t.shape[1] for t, _ in pairs]
    per_w = B // _NW
    n = per_w // _CH
    mesh = plsc.VectorSubcoreMesh(core_axis_name="c", subcore_axis_name="s")

    NB = 4
    n_main = (n // NB) * NB
    scratch = []
    for D in Ds:
        scratch += [pltpu.VMEM((_CH,), jnp.int32)] * NB
        scratch += [pltpu.VMEM((_CH, D), _F32)] * NB
        scratch += [pltpu.SemaphoreType.DMA] * (3 * NB)

    @functools.partial(
        pl.kernel,
        out_type=[jax.ShapeDtypeStruct((B, D), _F32) for D in Ds],
        mesh=mesh,
        scratch_types=scratch,
    )
    def k(*refs):
        tabs = [refs[2 * p] for p in range(kp)]
        idxs = [refs[2 * p + 1] for p in range(kp)]
        outs = [refs[2 * kp + p] for p in range(kp)]
        sc0 = 3 * kp
        PW = 5 * NB
        ib = [refs[sc0 + PW * p: sc0 + PW * p + NB] for p in range(kp)]
        rb = [refs[sc0 + PW * p + NB: sc0 + PW * p + 2 * NB] for p in range(kp)]
        sa = [refs[sc0 + PW * p + 2 * NB: sc0 + PW * p + 3 * NB] for p in range(kp)]
        sb = [refs[sc0 + PW * p + 3 * NB: sc0 + PW * p + 4 * NB] for p in range(kp)]
        so = [refs[sc0 + PW * p + 4 * NB: sc0 + PW * p + 5 * NB] for p in range(kp)]

        wid = lax.axis_index("s") * _NC + lax.axis_index("c")
        base = wid * per_w

        def drain_idx(p, b):
            pltpu.make_async_copy(idxs[p].at[pl.ds(base, _CH)],
                                  ib[p][b], sa[p][b]).wait()

        def drain_rows(p, b):
            pltpu.make_async_copy(tabs[p].at[pl.ds(0, _CH)],
                                  rb[p][b], sb[p][b]).wait()

        def drain_out(p, b):
            pltpu.make_async_copy(tabs[p].at[pl.ds(0, _CH)],
                                  rb[p][b], so[p][b]).wait()

        if n_main > 0:
            for p in range(kp):
                for b in range(NB):
                    pltpu.async_copy(idxs[p].at[pl.ds(base + b * _CH, _CH)],
                                     ib[p][b], sa[p][b])

            def body(g, _):
                for b in range(NB):
                    c = NB * g + b
                    for p in range(kp):
                        drain_idx(p, b)

                        @pl.when(c >= NB)
                        def _():
                            drain_out(p, b)

                        pltpu.async_copy(tabs[p].at[ib[p][b]], rb[p][b],
                                         sb[p][b])
                for b in range(NB):
                    c = NB * g + b
                    off = base + c * _CH
                    for p in range(kp):
                        drain_rows(p, b)
                        pltpu.async_copy(rb[p][b],
                                         outs[p].at[pl.ds(off, _CH)],
                                         so[p][b])

                        @pl.when(c + NB < n_main)
                        def _():
                            pltpu.async_copy(
                                idxs[p].at[pl.ds(off + NB * _CH, _CH)],
                                ib[p][b], sa[p][b])
                return 0

            lax.fori_loop(0, n_main // NB, body, 0)
            for p in range(kp):
                for b in range(NB):
                    drain_out(p, b)

        for c in range(n_main, n):
            off = base + c * _CH
            for p in range(kp):
                pltpu.sync_copy(idxs[p].at[pl.ds(off, _CH)], ib[p][0])
                pltpu.async_copy(tabs[p].at[ib[p][0]], rb[p][0],
                                 sb[p][0]).wait()
                pltpu.sync_copy(rb[p][0], outs[p].at[pl.ds(off, _CH)])

    flat = []
    for t, i in pairs:
        flat += [t, i]
    out = k(*flat)
    if not isinstance(out, (list, tuple)):
        out = [out]
    return list(out)


def _sc_gather(table, idx):
    return _sc_gather_multi([(table, idx)])[0]


def _sc_scatter_add(idx, vals, zeros_tab):
    B, D = vals.shape
    n_rows = zeros_tab.shape[0]
    per_w = B // _NW
    n_chunks = per_w // _CH
    rows_per_s = n_rows // _NS
    assert rows_per_s % 8 == 0 and n_rows % _NS == 0
    mesh = plsc.VectorSubcoreMesh(core_axis_name="c", subcore_axis_name="s")

    @functools.partial(
        pl.kernel,
        out_type=jax.ShapeDtypeStruct((2, n_rows, D), _F32),
        mesh=mesh,
        scratch_types=[pltpu.VMEM((_CH,), jnp.int32)] * 4 +
                      [pltpu.VMEM((_CH, D), _F32)] * 4 +
                      [pltpu.VMEM_SHARED((n_rows, D), _F32)] +
                      [pltpu.SemaphoreType.DMA] * 8,
    )
    def k(idx_hbm, vals_hbm, zeros_hbm, out_hbm, i0, i1, i2, i3,
          v0, v1, v2, v3, acc_sh, sa0, sa1, sa2, sa3, sv0, sv1, sv2, sv3):
        c = lax.axis_index("c")
        s = lax.axis_index("s")
        wid = s * _NC + c
        base = wid * per_w
        ib = (i0, i1, i2, i3)
        vb = (v0, v1, v2, v3)
        sa = (sa0, sa1, sa2, sa3)
        sv = (sv0, sv1, sv2, sv3)
        NB = 4

        @pl.when(s == 0)
        def _():
            pltpu.sync_copy(zeros_hbm, acc_sh)

        plsc.subcore_barrier()

        n_main = (n_chunks // NB) * NB
        if n_main > 0:
            for b in range(NB):
                off = base + b * _CH
                pltpu.async_copy(idx_hbm.at[pl.ds(off, _CH)], ib[b], sa[b])
                pltpu.async_copy(vals_hbm.at[pl.ds(off, _CH)], vb[b], sv[b])

            def body(g, _):
                for b in range(NB):
                    ch = NB * g + b
                    off = base + ch * _CH
                    pltpu.make_async_copy(idx_hbm.at[pl.ds(base, _CH)],
                                          ib[b], sa[b]).wait()
                    pltpu.make_async_copy(vals_hbm.at[pl.ds(base, _CH)],
                                          vb[b], sv[b]).wait()
                    pltpu.sync_copy(vb[b], acc_sh.at[ib[b]], add=True)

                    @pl.when(ch + NB < n_main)
                    def _():
                        pltpu.async_copy(
                            idx_hbm.at[pl.ds(off + NB * _CH, _CH)],
                            ib[b], sa[b])
                        pltpu.async_copy(
                            vals_hbm.at[pl.ds(off + NB * _CH, _CH)],
                            vb[b], sv[b])
                return 0

            lax.fori_loop(0, n_main // NB, body, 0)

        for ch in range(n_main, n_chunks):
            off = base + ch * _CH
            pltpu.sync_copy(idx_hbm.at[pl.ds(off, _CH)], ib[0])
            pltpu.sync_copy(vals_hbm.at[pl.ds(off, _CH)], vb[0])
            pltpu.sync_copy(vb[0], acc_sh.at[ib[0]], add=True)

        plsc.subcore_barrier()
        r0 = s * rows_per_s
        pltpu.sync_copy(acc_sh.at[pl.ds(r0, rows_per_s)],
                        out_hbm.at[c, pl.ds(r0, rows_per_s)])

    return k(idx, vals, zeros_tab)


def kernel(x, edge_index, edge_weight, emb, mlp_W1, mlp_b1, mlp_W2, mlp_b2,
           lin1_W, lin2_W, lin2_b, out1_W, out1_b, out2_W, out2_b):
    i_idx = edge_index[0].astype(jnp.int32)
    j_idx = edge_index[1].astype(jnp.int32)
    ew = edge_weight.astype(_F32)

    xb = jnp.broadcast_to(x.astype(_F32)[:, None], (_N, _H))
    embp = jnp.pad(emb.astype(_F32), ((0, _H - emb.shape[0]), (0, 0)))
    h, q = _tc_h0q0(xb, embp, lin1_W[0])

    W1p = jnp.pad(mlp_W1.astype(_F32), ((0, 0), (0, _NGP - _NG), (0, 0)))
    _NP = 10240
    zerosN = jnp.zeros((_NP, _H), _F32)

    qs = []
    ts = []
    for l in range(3):
        if l > 0:
            q = _tc_q(h, lin1_W[l])
        p = _sc_gather(q, j_idx)
        msg = _tc_edge_fwd(ew, p, W1p[l], mlp_b1[l].reshape(1, _H),
                           mlp_W2[l], mlp_b2[l].reshape(1, _H))
        agg2 = _sc_scatter_add(i_idx, msg, zerosN)
        h, t = _tc_node_fwd(h, agg2[0, :_N], agg2[1, :_N], lin2_W[l],
                            lin2_b[l].reshape(1, _H))
        qs.append(q)
        ts.append(t)

    G, e_acc = _tc_readout(h, out1_W, out1_b, out2_W, out2_b)
    total_energy = e_acc[0, 0]

    diff = jnp.zeros((_E, 3), _F32)
    for l in (2, 1, 0):
        dagg = _tc_node_bwd_pre(G, ts[l], lin2_W[l])
        dmsg, p = _sc_gather_multi([(dagg, i_idx), (qs[l], j_idx)])
        outs = _tc_edge_bwd(ew, p, dmsg, diff, W1p[l],
                            mlp_b1[l].reshape(1, _H), mlp_W2[l],
                            mlp_b2[l].reshape(1, _H), final=(l == 0))
        if l > 0:
            dp, diff = outs
            dh2 = _sc_scatter_add(j_idx, dp, zerosN)
            G = _tc_node_bwd_post(G, dh2[0, :_N], dh2[1, :_N], lin1_W[l])
        else:
            dp, pos, neg = outs

    vals = jnp.concatenate([pos, neg], axis=0)
    sidx = jnp.concatenate([i_idx, j_idx], axis=0)
    f2 = _sc_scatter_add(sidx, vals, zerosN)
    forces = _tc_combine(f2[0, :_N], f2[1, :_N])[:, :3]
    return (total_energy, forces)


# speedup vs baseline: 1.6269x; 1.1560x over previous
import functools

import numpy as np
import jax
import jax.numpy as jnp
from jax import lax
from jax.experimental import pallas as pl
from jax.experimental.pallas import tpu as pltpu
from jax.experimental.pallas import tpu_sc as plsc

_N = 10000
_E = 320000
_H = 128
_NG = 50
_NGP = 64
_CUT = 5.0
_LOG2 = float(np.log(2.0))
_COEFF = -0.5 / (_CUT / _NG) ** 2
_PI = float(np.pi)

def _offsets():
    col = lax.broadcasted_iota(jnp.int32, (1, _NGP), 1)
    colf = col.astype(_F32)
    return jnp.where(col < _NG, colf * (_CUT / (_NG - 1)), 0.0)

_NC = 2
_NS = 16
_NW = _NC * _NS
_CH = 80

_F32 = jnp.float32


def _ssp(v):
    return jax.nn.softplus(v) - _LOG2


def _edge_geom(ew):
    d = jnp.sqrt(jnp.sum(ew * ew, axis=1, keepdims=True))
    offs = _offsets()
    rbf = jnp.exp(_COEFF * (d - offs) ** 2)
    C = 0.5 * (jnp.cos(d * (_PI / _CUT)) + 1.0) * (d <= _CUT)
    return d, rbf, C


def _mm(a, b):
    return jnp.dot(a.astype(jnp.bfloat16), b.astype(jnp.bfloat16),
                   preferred_element_type=_F32)


def _mm_t(a, b):
    return lax.dot_general(a.astype(jnp.bfloat16), b.astype(jnp.bfloat16),
                           (((1,), (1,)), ((), ())),
                           preferred_element_type=_F32)


def _q_body(h_ref, w_ref, o_ref):
    o_ref[...] = _mm(h_ref[...], w_ref[...])


def _tc_q(h, w):
    BN = 2000
    return pl.pallas_call(
        _q_body,
        grid=(_N // BN,),
        in_specs=[pl.BlockSpec((BN, _H), lambda b: (b, 0)),
                  pl.BlockSpec((_H, _H), lambda b: (0, 0))],
        out_specs=pl.BlockSpec((BN, _H), lambda b: (b, 0)),
        out_shape=jax.ShapeDtypeStruct((_N, _H), _F32),
    )(h, w)


def _edge_fwd_body(ew_ref, p_ref, w1_ref, b1_ref, w2_ref, b2_ref, msg_ref):
    d, rbf, C = _edge_geom(ew_ref[...])
    A = _mm(rbf, w1_ref[...]) + b1_ref[...]
    B = _mm(_ssp(A), w2_ref[...]) + b2_ref[...]
    msg_ref[...] = (B * C) * p_ref[...]


def _tc_edge_fwd(ew, p, w1, b1, w2, b2):
    BE = 1600
    g = _E // BE
    return pl.pallas_call(
        _edge_fwd_body,
        grid=(g,),
        in_specs=[pl.BlockSpec((BE, 3), lambda b: (b, 0)),
                  pl.BlockSpec((BE, _H), lambda b: (b, 0)),
                  pl.BlockSpec((_NGP, _H), lambda b: (0, 0)),
                  pl.BlockSpec((1, _H), lambda b: (0, 0)),
                  pl.BlockSpec((_H, _H), lambda b: (0, 0)),
                  pl.BlockSpec((1, _H), lambda b: (0, 0))],
        out_specs=pl.BlockSpec((BE, _H), lambda b: (b, 0)),
        out_shape=jax.ShapeDtypeStruct((_E, _H), _F32),
    )(ew, p, w1, b1, w2, b2)


def _node_fwd_body(h_ref, a0_ref, a1_ref, w_ref, b_ref, hn_ref, t_ref):
    t = _mm(a0_ref[...] + a1_ref[...], w_ref[...]) + b_ref[...]
    t_ref[...] = t
    hn_ref[...] = h_ref[...] + _ssp(t)


def _tc_node_fwd(h, a0, a1, w, b):
    BN = 2000
    blk = pl.BlockSpec((BN, _H), lambda i: (i, 0))
    wblk = pl.BlockSpec((_H, _H), lambda i: (0, 0))
    bblk = pl.BlockSpec((1, _H), lambda i: (0, 0))
    return pl.pallas_call(
        _node_fwd_body,
        grid=(_N // BN,),
        in_specs=[blk, blk, blk, wblk, bblk],
        out_specs=[blk, blk],
        out_shape=[jax.ShapeDtypeStruct((_N, _H), _F32),
                   jax.ShapeDtypeStruct((_N, _H), _F32)],
    )(h, a0, a1, w, b)


def _readout_body(h_ref, w1_ref, b1_ref, w2r_ref, b2_ref, g_ref, e_ref):
    h = h_ref[...]
    y1 = _mm(h, w1_ref[...]) + b1_ref[...]
    w2r = w2r_ref[...]
    zb = _ssp(y1).astype(jnp.bfloat16).astype(_F32)
    wb = w2r.astype(jnp.bfloat16).astype(_F32)
    s = jnp.sum(zb * wb) + h.shape[0] * b2_ref[0, 0]

    @pl.when(pl.program_id(0) == 0)
    def _():
        e_ref[...] = jnp.zeros_like(e_ref)

    e_ref[...] += jnp.full(e_ref.shape, s, _F32)
    g_ref[...] = _mm_t(jax.nn.sigmoid(y1) * w2r, w1_ref[...])


def _tc_readout(h, out1_W, out1_b, out2_W, out2_b):
    BN = 2000
    return pl.pallas_call(
        _readout_body,
        grid=(_N // BN,),
        in_specs=[pl.BlockSpec((BN, _H), lambda i: (i, 0)),
                  pl.BlockSpec((_H, 64), lambda i: (0, 0)),
                  pl.BlockSpec((1, 64), lambda i: (0, 0)),
                  pl.BlockSpec((1, 64), lambda i: (0, 0)),
                  pl.BlockSpec((1, 128), lambda i: (0, 0))],
        out_specs=[pl.BlockSpec((BN, _H), lambda i: (i, 0)),
                   pl.BlockSpec((8, 128), lambda i: (0, 0))],
        out_shape=[jax.ShapeDtypeStruct((_N, _H), _F32),
                   jax.ShapeDtypeStruct((8, 128), _F32)],
    )(h, out1_W, out1_b.reshape(1, 64), out2_W.reshape(1, 64),
      jnp.broadcast_to(out2_b.reshape(1, 1), (1, 128)))


def _bwd_pre_body(g_ref, t_ref, w_ref, o_ref):
    o_ref[...] = _mm_t(g_ref[...] * jax.nn.sigmoid(t_ref[...]), w_ref[...])


def _tc_node_bwd_pre(G, t, w):
    BN = 2000
    blk = pl.BlockSpec((BN, _H), lambda i: (i, 0))
    return pl.pallas_call(
        _bwd_pre_body,
        grid=(_N // BN,),
        in_specs=[blk, blk, pl.BlockSpec((_H, _H), lambda i: (0, 0))],
        out_specs=blk,
        out_shape=jax.ShapeDtypeStruct((_N, _H), _F32),
    )(G, t, w)


def _edge_bwd_common(ew_ref, p_ref, dm_ref, dprev_ref,
                     w1_ref, b1_ref, w2_ref, b2_ref, dp_ref):
    ew = ew_ref[...]
    d, rbf, C = _edge_geom(ew)
    w1 = w1_ref[...]
    w2 = w2_ref[...]
    A = _mm(rbf, w1) + b1_ref[...]
    sigA = jax.nn.sigmoid(A)
    B = _mm(_ssp(A), w2) + b2_ref[...]
    Wf = B * C
    dm = dm_ref[...]
    p = p_ref[...]
    dWf = dm * p
    dp_ref[...] = dm * Wf
    dC = jnp.sum(dWf * B, axis=1, keepdims=True)
    dS = _mm_t(dWf * C, w2)
    drbf = _mm_t(dS * sigA, w1)
    offs = _offsets()
    ddr = jnp.sum(drbf * rbf * (2.0 * _COEFF) * (d - offs),
                  axis=1, keepdims=True)
    dCd = (-0.5 * _PI / _CUT) * jnp.sin(d * (_PI / _CUT)) * (d <= _CUT)
    dd = ddr + dC * dCd
    return dprev_ref[...] + dd * ew / d


def _edge_bwd_body(ew_ref, p_ref, dm_ref, dprev_ref,
                   w1_ref, b1_ref, w2_ref, b2_ref, dp_ref, dout_ref):
    dout_ref[...] = _edge_bwd_common(ew_ref, p_ref, dm_ref, dprev_ref,
                                     w1_ref, b1_ref, w2_ref, b2_ref, dp_ref)


def _edge_bwd_final_body(ew_ref, p_ref, dm_ref, dprev_ref,
                         w1_ref, b1_ref, w2_ref, b2_ref,
                         dp_ref, pos_ref, neg_ref):
    dnew = _edge_bwd_common(ew_ref, p_ref, dm_ref, dprev_ref,
                            w1_ref, b1_ref, w2_ref, b2_ref, dp_ref)
    z = jnp.zeros((dnew.shape[0], _H - 3), _F32)
    pos = jnp.concatenate([dnew, z], axis=1)
    pos_ref[...] = pos
    neg_ref[...] = -pos


def _tc_edge_bwd(ew, p, dm, dprev, w1, b1, w2, b2, final):
    BE = 1600
    g = _E // BE
    eblk = pl.BlockSpec((BE, _H), lambda b: (b, 0))
    vblk = pl.BlockSpec((BE, 3), lambda b: (b, 0))
    if final:
        body = _edge_bwd_final_body
        out_specs = [eblk, eblk, eblk]
        out_shape = [jax.ShapeDtypeStruct((_E, _H), _F32),
                     jax.ShapeDtypeStruct((_E, _H), _F32),
                     jax.ShapeDtypeStruct((_E, _H), _F32)]
    else:
        body = _edge_bwd_body
        out_specs = [eblk, vblk]
        out_shape = [jax.ShapeDtypeStruct((_E, _H), _F32),
                     jax.ShapeDtypeStruct((_E, 3), _F32)]
    return pl.pallas_call(
        body,
        grid=(g,),
        in_specs=[vblk, eblk, eblk, vblk,
                  pl.BlockSpec((_NGP, _H), lambda b: (0, 0)),
                  pl.BlockSpec((1, _H), lambda b: (0, 0)),
                  pl.BlockSpec((_H, _H), lambda b: (0, 0)),
                  pl.BlockSpec((1, _H), lambda b: (0, 0))],
        out_specs=out_specs,
        out_shape=out_shape,
    )(ew, p, dm, dprev, w1, b1, w2, b2)


def _h0q0_body(xb_ref, emb_ref, w_ref, h_ref, q_ref):
    lane = lax.broadcasted_iota(jnp.int32, xb_ref.shape, 1).astype(_F32)
    onehot = (xb_ref[...] == lane).astype(_F32)
    h = jnp.dot(onehot, emb_ref[...], preferred_element_type=_F32,
                precision=lax.Precision.HIGHEST)
    h_ref[...] = h
    q_ref[...] = _mm(h, w_ref[...])


def _tc_h0q0(xb, embp, w):
    BN = 2000
    blk = pl.BlockSpec((BN, _H), lambda i: (i, 0))
    wblk = pl.BlockSpec((_H, _H), lambda i: (0, 0))
    return pl.pallas_call(
        _h0q0_body,
        grid=(_N // BN,),
        in_specs=[blk, wblk, wblk],
        out_specs=[blk, blk],
        out_shape=[jax.ShapeDtypeStruct((_N, _H), _F32),
                   jax.ShapeDtypeStruct((_N, _H), _F32)],
    )(xb, embp, w)


def _bwd_post_body(g_ref, a0_ref, a1_ref, w_ref, o_ref):
    o_ref[...] = g_ref[...] + _mm_t(a0_ref[...] + a1_ref[...], w_ref[...])


def _tc_node_bwd_post(G, a0, a1, w):
    BN = 2000
    blk = pl.BlockSpec((BN, _H), lambda i: (i, 0))
    return pl.pallas_call(
        _bwd_post_body,
        grid=(_N // BN,),
        in_specs=[blk, blk, blk, pl.BlockSpec((_H, _H), lambda i: (0, 0))],
        out_specs=blk,
        out_shape=jax.ShapeDtypeStruct((_N, _H), _F32),
    )(G, a0, a1, w)


def _combine_body(a_ref, b_ref, o_ref):
    o_ref[...] = a_ref[...] + b_ref[...]


def _tc_combine(a, b):
    BN = 2000
    D = a.shape[1]
    blk = pl.BlockSpec((BN, D), lambda i: (i, 0))
    return pl.pallas_call(
        _combine_body,
        grid=(_N // BN,),
        in_specs=[blk, blk],
        out_specs=blk,
        out_shape=jax.ShapeDtypeStruct((_N, D), _F32),
    )(a, b)


def _sc_gather_multi(pairs):
    kp = len(pairs)
    B = pairs[0][1].shape[0]
    Ds = [t.shape[1] for t, _ in pairs]
    per_w = B // _NW
    n = per_w // _CH
    mesh = plsc.VectorSubcoreMesh(core_axis_name="c", subcore_axis_name="s")

    NB = 4
    n_main = (n // NB) * NB
    scratch = []
    for D in Ds:
        scratch += [pltpu.VMEM((_CH,), jnp.int32)] * NB
        scratch += [pltpu.VMEM((_CH, D), _F32)] * NB
        scratch += [pltpu.SemaphoreType.DMA] * (3 * NB)

    @functools.partial(
        pl.kernel,
        out_type=[jax.ShapeDtypeStruct((B, D), _F32) for D in Ds],
        mesh=mesh,
        scratch_types=scratch,
    )
    def k(*refs):
        tabs = [refs[2 * p] for p in range(kp)]
        idxs = [refs[2 * p + 1] for p in range(kp)]
        outs = [refs[2 * kp + p] for p in range(kp)]
        sc0 = 3 * kp
        PW = 5 * NB
        ib = [refs[sc0 + PW * p: sc0 + PW * p + NB] for p in range(kp)]
        rb = [refs[sc0 + PW * p + NB: sc0 + PW * p + 2 * NB] for p in range(kp)]
        sa = [refs[sc0 + PW * p + 2 * NB: sc0 + PW * p + 3 * NB] for p in range(kp)]
        sb = [refs[sc0 + PW * p + 3 * NB: sc0 + PW * p + 4 * NB] for p in range(kp)]
        so = [refs[sc0 + PW * p + 4 * NB: sc0 + PW * p + 5 * NB] for p in range(kp)]

        wid = lax.axis_index("s") * _NC + lax.axis_index("c")
        base = wid * per_w

        def drain_idx(p, b):
            pltpu.make_async_copy(idxs[p].at[pl.ds(base, _CH)],
                                  ib[p][b], sa[p][b]).wait()

        def drain_rows(p, b):
            pltpu.make_async_copy(tabs[p].at[pl.ds(0, _CH)],
                                  rb[p][b], sb[p][b]).wait()

        def drain_out(p, b):
            pltpu.make_async_copy(tabs[p].at[pl.ds(0, _CH)],
                                  rb[p][b], so[p][b]).wait()

        if n_main > 0:
            for p in range(kp):
                for b in range(NB):
                    pltpu.async_copy(idxs[p].at[pl.ds(base + b * _CH, _CH)],
                                     ib[p][b], sa[p][b])

            def body(g, _):
                for b in range(NB):
                    c = NB * g + b
                    for p in range(kp):
                        drain_idx(p, b)

                        @pl.when(c >= NB)
                        def _():
                            drain_out(p, b)

                        pltpu.async_copy(tabs[p].at[ib[p][b]], rb[p][b],
                                         sb[p][b])
                for b in range(NB):
                    c = NB * g + b
                    off = base + c * _CH
                    for p in range(kp):
                        drain_rows(p, b)
                        pltpu.async_copy(rb[p][b],
                                         outs[p].at[pl.ds(off, _CH)],
                                         so[p][b])

                        @pl.when(c + NB < n_main)
                        def _():
                            pltpu.async_copy(
                                idxs[p].at[pl.ds(off + NB * _CH, _CH)],
                                ib[p][b], sa[p][b])
                return 0

            lax.fori_loop(0, n_main // NB, body, 0)
            for p in range(kp):
                for b in range(NB):
                    drain_out(p, b)

        for c in range(n_main, n):
            off = base + c * _CH
            for p in range(kp):
                pltpu.sync_copy(idxs[p].at[pl.ds(off, _CH)], ib[p][0])
                pltpu.async_copy(tabs[p].at[ib[p][0]], rb[p][0],
                                 sb[p][0]).wait()
                pltpu.sync_copy(rb[p][0], outs[p].at[pl.ds(off, _CH)])

    flat = []
    for t, i in pairs:
        flat += [t, i]
    out = k(*flat)
    if not isinstance(out, (list, tuple)):
        out = [out]
    return list(out)


def _sc_gather(table, idx):
    return _sc_gather_multi([(table, idx)])[0]


def _sc_scatter_add(idx, vals, zeros_tab):
    B, D = vals.shape
    n_rows = zeros_tab.shape[0]
    per_w = B // _NW
    n_chunks = per_w // _CH
    rows_per_s = n_rows // _NS
    assert rows_per_s % 8 == 0 and n_rows % _NS == 0
    mesh = plsc.VectorSubcoreMesh(core_axis_name="c", subcore_axis_name="s")

    @functools.partial(
        pl.kernel,
        out_type=jax.ShapeDtypeStruct((2, n_rows, D), _F32),
        mesh=mesh,
        scratch_types=[pltpu.VMEM((_CH,), jnp.int32)] * 4 +
                      [pltpu.VMEM((_CH, D), _F32)] * 4 +
                      [pltpu.VMEM_SHARED((n_rows, D), _F32)] +
                      [pltpu.SemaphoreType.DMA] * 8,
    )
    def k(idx_hbm, vals_hbm, zeros_hbm, out_hbm, i0, i1, i2, i3,
          v0, v1, v2, v3, acc_sh, sa0, sa1, sa2, sa3, sv0, sv1, sv2, sv3):
        c = lax.axis_index("c")
        s = lax.axis_index("s")
        wid = s * _NC + c
        base = wid * per_w
        ib = (i0, i1, i2, i3)
        vb = (v0, v1, v2, v3)
        sa = (sa0, sa1, sa2, sa3)
        sv = (sv0, sv1, sv2, sv3)
        NB = 4

        @pl.when(s == 0)
        def _():
            pltpu.sync_copy(zeros_hbm, acc_sh)

        plsc.subcore_barrier()

        n_main = (n_chunks // NB) * NB
        if n_main > 0:
            for b in range(NB):
                off = base + b * _CH
                pltpu.async_copy(idx_hbm.at[pl.ds(off, _CH)], ib[b], sa[b])
                pltpu.async_copy(vals_hbm.at[pl.ds(off, _CH)], vb[b], sv[b])

            def body(g, _):
                for b in range(NB):
                    ch = NB * g + b
                    off = base + ch * _CH
                    pltpu.make_async_copy(idx_hbm.at[pl.ds(base, _CH)],
                                          ib[b], sa[b]).wait()
                    pltpu.make_async_copy(vals_hbm.at[pl.ds(base, _CH)],
                                          vb[b], sv[b]).wait()
                    pltpu.sync_copy(vb[b], acc_sh.at[ib[b]], add=True)

                    @pl.when(ch + NB < n_main)
                    def _():
                        pltpu.async_copy(
                            idx_hbm.at[pl.ds(off + NB * _CH, _CH)],
                            ib[b], sa[b])
                        pltpu.async_copy(
                            vals_hbm.at[pl.ds(off + NB * _CH, _CH)],
                            vb[b], sv[b])
                return 0

            lax.fori_loop(0, n_main // NB, body, 0)

        for ch in range(n_main, n_chunks):
            off = base + ch * _CH
            pltpu.sync_copy(idx_hbm.at[pl.ds(off, _CH)], ib[0])
            pltpu.sync_copy(vals_hbm.at[pl.ds(off, _CH)], vb[0])
            pltpu.sync_copy(vb[0], acc_sh.at[ib[0]], add=True)

        plsc.subcore_barrier()
        r0 = s * rows_per_s
        pltpu.sync_copy(acc_sh.at[pl.ds(r0, rows_per_s)],
                        out_hbm.at[c, pl.ds(r0, rows_per_s)])

    return k(idx, vals, zeros_tab)


def kernel(x, edge_index, edge_weight, emb, mlp_W1, mlp_b1, mlp_W2, mlp_b2,
           lin1_W, lin2_W, lin2_b, out1_W, out1_b, out2_W, out2_b):
    i_idx = edge_index[0].astype(jnp.int32)
    j_idx = edge_index[1].astype(jnp.int32)
    ew = edge_weight.astype(_F32)

    xb = jnp.broadcast_to(x.astype(_F32)[:, None], (_N, _H))
    embp = jnp.pad(emb.astype(_F32), ((0, _H - emb.shape[0]), (0, 0)))
    h, q = _tc_h0q0(xb, embp, lin1_W[0])

    W1p = jnp.pad(mlp_W1.astype(_F32), ((0, 0), (0, _NGP - _NG), (0, 0)))
    _NP = 10240
    zerosN = jnp.zeros((_NP, _H), _F32)

    qs = []
    ts = []
    for l in range(3):
        if l > 0:
            q = _tc_q(h, lin1_W[l])
        p = _sc_gather(q, j_idx)
        msg = _tc_edge_fwd(ew, p, W1p[l], mlp_b1[l].reshape(1, _H),
                           mlp_W2[l], mlp_b2[l].reshape(1, _H))
        agg2 = _sc_scatter_add(i_idx, msg, zerosN)
        h, t = _tc_node_fwd(h, agg2[0, :_N], agg2[1, :_N], lin2_W[l],
                            lin2_b[l].reshape(1, _H))
        qs.append(q)
        ts.append(t)

    G, e_acc = _tc_readout(h, out1_W, out1_b, out2_W, out2_b)
    total_energy = e_acc[0, 0]

    diff = jnp.zeros((_E, 3), _F32)
    for l in (2, 1, 0):
        dagg = _tc_node_bwd_pre(G, ts[l], lin2_W[l])
        dmsg, p = _sc_gather_multi([(dagg, i_idx), (qs[l], j_idx)])
        outs = _tc_edge_bwd(ew, p, dmsg, diff, W1p[l],
                            mlp_b1[l].reshape(1, _H), mlp_W2[l],
                            mlp_b2[l].reshape(1, _H), final=(l == 0))
        if l > 0:
            dp, diff = outs
            dh2 = _sc_scatter_add(j_idx, dp, zerosN)
            G = _tc_node_bwd_post(G, dh2[0, :_N], dh2[1, :_N], lin1_W[l])
        else:
            dp, pos, neg = outs

    vals = jnp.concatenate([pos, neg], axis=0)
    sidx = jnp.concatenate([i_idx, j_idx], axis=0)
    f2 = _sc_scatter_add(sidx, vals, zerosN)
    forces = _tc_combine(f2[0, :_N], f2[1, :_N])[:, :3]
    return (total_energy, forces)
